# baseline (speedup 1.0000x reference)
import jax
import jax.numpy as jnp
from jax.experimental import pallas as pl
from jax.experimental.pallas import tpu as pltpu

_COST_MASK = 5.0
_COST_DICE = 5.0
_COST_BOX = 5.0
_COST_GIOU = 2.0

_NB = 128
_NPB = 6
_N2 = _NB * _NPB
_QP = 384
_U = 8
_KC = 8


def _sample_kernel(idx_ref, wx_ref, wy_ref, src_hbm, out_ref, buf, sem, *, wp, n_per_batch):
    g = pl.program_id(0)
    cp = pltpu.make_async_copy(src_hbm.at[g], buf, sem)
    cp.start()
    cp.wait()
    npts = out_ref.shape[0]

    def body(c, _):
        base = c * _U
        for u in range(_U):
            p = base + u
            i00 = idx_ref[0, p]
            wxs = wx_ref[0, p]
            wys = wy_ref[0, p]
            v00 = buf[i00, 0]
            v01 = buf[i00 + 1, 0]
            v10 = buf[i00 + wp, 0]
            v11 = buf[i00 + wp + 1, 0]
            r0 = v00 + wxs * (v01 - v00)
            r1 = v10 + wxs * (v11 - v10)
            out_ref[p, 0] = r0 + wys * (r1 - r0)
        return ()

    jax.lax.fori_loop(0, npts // _U, body, ())


def _cost_kernel(samp_ref, pb_ref, tbt_ref, out_ref, a_acc, sd_acc, neg_acc, tms_acc):
    k = pl.program_id(1)
    pc = samp_ref.shape[0]

    @pl.when(k == 0)
    def _():
        a_acc[...] = jnp.zeros_like(a_acc)
        sd_acc[...] = jnp.zeros_like(sd_acc)
        neg_acc[...] = jnp.zeros_like(neg_acc)
        tms_acc[...] = jnp.zeros_like(tms_acc)

    om = samp_ref[:, 0:_QP]
    tm = samp_ref[:, _QP:2 * _QP]
    e = jnp.exp(-jnp.abs(om))
    s_abs = 1.0 / (1.0 + e)
    s = jnp.where(om >= 0.0, s_abs, 1.0 - s_abs)
    neg = jnp.maximum(om, 0.0) + jnp.log(1.0 + e)

    ones = jnp.ones((pc, _NB), jnp.float32)
    rhs = jnp.concatenate([tm, ones], axis=1)
    dn = (((0,), (0,)), ((), ()))
    a_acc[...] += jax.lax.dot_general(om, tm, dn, preferred_element_type=jnp.float32)
    sd_acc[...] += jax.lax.dot_general(s, rhs, dn, preferred_element_type=jnp.float32)
    neg_acc[...] += jax.lax.dot_general(neg, ones, dn, preferred_element_type=jnp.float32)
    tms_acc[...] += jax.lax.dot_general(ones, tm, dn, preferred_element_type=jnp.float32)

    @pl.when(k == _KC - 1)
    def _():
        p_total = float(pc * _KC)
        a = a_acc[...]
        sd = sd_acc[:, 0:_QP]
        ssum = pltpu.repeat(sd_acc[:, _QP:_QP + _NB], _QP // _NB, axis=1)
        negs = pltpu.repeat(neg_acc[...], _QP // _NB, axis=1)
        tms = pltpu.repeat(tms_acc[...], _QP // _NB, axis=0)

        cost_mask = (negs - a) * (1.0 / p_total)
        cost_dice = 1.0 - (2.0 * sd + 1.0) / (ssum + tms + 1.0)

        pcx, pcy = pb_ref[:, 0:1], pb_ref[:, 1:2]
        pw, ph = pb_ref[:, 2:3], pb_ref[:, 3:4]
        tcx, tcy = tbt_ref[0:1, :], tbt_ref[1:2, :]
        tw, th = tbt_ref[2:3, :], tbt_ref[3:4, :]

        l1 = (jnp.abs(pcx - tcx) + jnp.abs(pcy - tcy)
              + jnp.abs(pw - tw) + jnp.abs(ph - th))

        ax1, ax2 = pcx - 0.5 * pw, pcx + 0.5 * pw
        ay1, ay2 = pcy - 0.5 * ph, pcy + 0.5 * ph
        bx1, bx2 = tcx - 0.5 * tw, tcx + 0.5 * tw
        by1, by2 = tcy - 0.5 * th, tcy + 0.5 * th
        area_a = (ax2 - ax1) * (ay2 - ay1)
        area_b = (bx2 - bx1) * (by2 - by1)
        iw = jnp.maximum(jnp.minimum(ax2, bx2) - jnp.maximum(ax1, bx1), 0.0)
        ih = jnp.maximum(jnp.minimum(ay2, by2) - jnp.maximum(ay1, by1), 0.0)
        inter = iw * ih
        union = area_a + area_b - inter
        iou = inter / union
        ew = jnp.maximum(bx2, ax2) - jnp.minimum(bx1, ax1)
        eh = jnp.maximum(by2, ay2) - jnp.minimum(by1, ay1)
        area_e = ew * eh
        giou = iou - (area_e - union) / area_e

        out_ref[...] = (_COST_MASK * cost_mask + _COST_DICE * cost_dice
                        + _COST_BOX * l1 - _COST_GIOU * giou)


def kernel(pred_masks, tgt_masks, pred_boxes, tgt_boxes, point_coords):
    B, Q, H, W = pred_masks.shape
    T = tgt_masks.shape[1]
    P = point_coords.shape[1]
    hp, wp = H + 2, W + 2
    hwp = hp * wp
    f32 = jnp.float32

    allm = jnp.zeros((B, _N2, hp, wp), f32)
    allm = allm.at[:, :Q, 1:H + 1, 1:W + 1].set(pred_masks)
    allm = allm.at[:, _QP:_QP + T, 1:H + 1, 1:W + 1].set(tgt_masks)
    src = (allm.reshape(B, _NPB, _NB, hwp)
               .transpose(0, 1, 3, 2)
               .reshape(B * _NPB, hwp, 1, _NB))

    px = point_coords[..., 0] * W - 0.5
    py = point_coords[..., 1] * H - 0.5
    x0 = jnp.floor(px)
    y0 = jnp.floor(py)
    wxa = px - x0
    wya = py - y0
    ix = x0.astype(jnp.int32) + 1
    iy = y0.astype(jnp.int32) + 1
    idx = (iy * wp + ix).reshape(B, 1, P)
    wxa = wxa.reshape(B, 1, P)
    wya = wya.reshape(B, 1, P)

    sampled = pl.pallas_call(
        lambda *refs: _sample_kernel(*refs, wp=wp, n_per_batch=_NPB),
        grid=(B * _NPB,),
        in_specs=[
            pl.BlockSpec((None, 1, P), lambda g: (g // _NPB, 0, 0), memory_space=pltpu.SMEM),
            pl.BlockSpec((None, 1, P), lambda g: (g // _NPB, 0, 0), memory_space=pltpu.SMEM),
            pl.BlockSpec((None, 1, P), lambda g: (g // _NPB, 0, 0), memory_space=pltpu.SMEM),
            pl.BlockSpec(memory_space=pl.ANY),
        ],
        out_specs=pl.BlockSpec((None, P, 1, _NB), lambda g: (g // _NPB, 0, 0, g % _NPB)),
        out_shape=jax.ShapeDtypeStruct((B, P, 1, _N2), f32),
        scratch_shapes=[
            pltpu.VMEM((hwp, 1, _NB), f32),
            pltpu.SemaphoreType.DMA,
        ],
        compiler_params=pltpu.CompilerParams(
            dimension_semantics=("parallel",),
            vmem_limit_bytes=56 * 1024 * 1024,
        ),
    )(idx, wxa, wya, src)

    samp = sampled.reshape(B, P, _N2)

    pb = jnp.zeros((B, _QP, 4), f32).at[:, :Q, :].set(pred_boxes)
    tbt = jnp.zeros((B, 4, _QP), f32).at[:, :, :T].set(tgt_boxes.transpose(0, 2, 1))

    pc = P // _KC
    out = pl.pallas_call(
        _cost_kernel,
        grid=(B, _KC),
        in_specs=[
            pl.BlockSpec((None, pc, _N2), lambda b, k: (b, k, 0)),
            pl.BlockSpec((None, _QP, 4), lambda b, k: (b, 0, 0)),
            pl.BlockSpec((None, 4, _QP), lambda b, k: (b, 0, 0)),
        ],
        out_specs=pl.BlockSpec((None, _QP, _QP), lambda b, k: (b, 0, 0)),
        out_shape=jax.ShapeDtypeStruct((B, _QP, _QP), f32),
        scratch_shapes=[
            pltpu.VMEM((_QP, _QP), f32),
            pltpu.VMEM((_QP, _QP + _NB), f32),
            pltpu.VMEM((_QP, _NB), f32),
            pltpu.VMEM((_NB, _QP), f32),
        ],
        compiler_params=pltpu.CompilerParams(
            dimension_semantics=("parallel", "arbitrary"),
            vmem_limit_bytes=56 * 1024 * 1024,
        ),
    )(samp, pb, tbt)

    return out[:, :Q, :T]


# speedup vs baseline: 3.3996x; 3.3996x over previous
import jax
import jax.numpy as jnp
from jax.experimental import pallas as pl
from jax.experimental.pallas import tpu as pltpu

_COST_MASK = 5.0
_COST_DICE = 5.0
_COST_BOX = 5.0
_COST_GIOU = 2.0

_NB = 128
_NPB = 6
_N2 = _NB * _NPB
_QP = 384
_U = 8
_KC = 8
_RH = 32


def _pack_kernel(x_ref, o_ref):
    w = x_ref.shape[2]
    ident = jnp.eye(128, dtype=jnp.float32)
    dn = (((0,), (0,)), ((), ()))
    for r in range(_RH):
        piece = x_ref[:, r, :]
        o_ref[r * w:(r + 1) * w, :] = jax.lax.dot_general(
            piece, ident, dn, preferred_element_type=jnp.float32)


def _sample_kernel(idx_ref, wx_ref, wy_ref, src_hbm, out_ref, buf, sem, *, h, w):
    g = pl.program_id(0)
    wp = w + 2
    zrow = jnp.zeros((_NB,), jnp.float32)
    buf[pl.ds(0, wp), 0, :] = jnp.zeros((wp, _NB), jnp.float32)
    buf[pl.ds((h + 1) * wp, wp), 0, :] = jnp.zeros((wp, _NB), jnp.float32)

    def zbody(y, _):
        buf[y * wp, 0, :] = zrow
        buf[y * wp + wp - 1, 0, :] = zrow
        return ()
    jax.lax.fori_loop(1, h + 1, zbody, ())

    for y in range(h):
        pltpu.make_async_copy(src_hbm.at[g, pl.ds(y * w, w)],
                              buf.at[pl.ds((y + 1) * wp + 1, w)], sem).start()
    for y in range(h):
        pltpu.make_async_copy(src_hbm.at[g, pl.ds(y * w, w)],
                              buf.at[pl.ds((y + 1) * wp + 1, w)], sem).wait()

    npts = out_ref.shape[0]

    def body(c, _):
        base = c * _U
        for u in range(_U):
            p = base + u
            i00 = idx_ref[0, p]
            wxs = wx_ref[0, p]
            wys = wy_ref[0, p]
            v00 = buf[i00, 0]
            v01 = buf[i00 + 1, 0]
            v10 = buf[i00 + wp, 0]
            v11 = buf[i00 + wp + 1, 0]
            r0 = v00 + wxs * (v01 - v00)
            r1 = v10 + wxs * (v11 - v10)
            out_ref[p, 0] = r0 + wys * (r1 - r0)
        return ()

    jax.lax.fori_loop(0, npts // _U, body, ())


def _cost_kernel(samp_ref, pb_ref, tbt_ref, out_ref, a_acc, sd_acc, neg_acc, tms_acc):
    k = pl.program_id(1)
    pc = samp_ref.shape[0]

    @pl.when(k == 0)
    def _():
        a_acc[...] = jnp.zeros_like(a_acc)
        sd_acc[...] = jnp.zeros_like(sd_acc)
        neg_acc[...] = jnp.zeros_like(neg_acc)
        tms_acc[...] = jnp.zeros_like(tms_acc)

    om = samp_ref[:, 0:_QP]
    tm = samp_ref[:, _QP:2 * _QP]
    e = jnp.exp(-jnp.abs(om))
    s_abs = 1.0 / (1.0 + e)
    s = jnp.where(om >= 0.0, s_abs, 1.0 - s_abs)
    neg = jnp.maximum(om, 0.0) + jnp.log(1.0 + e)

    ones = jnp.ones((pc, _NB), jnp.float32)
    rhs = jnp.concatenate([tm, ones], axis=1)
    dn = (((0,), (0,)), ((), ()))
    a_acc[...] += jax.lax.dot_general(om, tm, dn, preferred_element_type=jnp.float32)
    sd_acc[...] += jax.lax.dot_general(s, rhs, dn, preferred_element_type=jnp.float32)
    neg_acc[...] += jax.lax.dot_general(neg, ones, dn, preferred_element_type=jnp.float32)
    tms_acc[...] += jax.lax.dot_general(ones, tm, dn, preferred_element_type=jnp.float32)

    @pl.when(k == _KC - 1)
    def _():
        p_total = float(pc * _KC)
        a = a_acc[...]
        sd = sd_acc[:, 0:_QP]
        ssum = pltpu.repeat(sd_acc[:, _QP:_QP + _NB], _QP // _NB, axis=1)
        negs = pltpu.repeat(neg_acc[...], _QP // _NB, axis=1)
        tms = pltpu.repeat(tms_acc[...], _QP // _NB, axis=0)

        cost_mask = (negs - a) * (1.0 / p_total)
        cost_dice = 1.0 - (2.0 * sd + 1.0) / (ssum + tms + 1.0)

        pcx, pcy = pb_ref[:, 0:1], pb_ref[:, 1:2]
        pw, ph = pb_ref[:, 2:3], pb_ref[:, 3:4]
        tcx, tcy = tbt_ref[0:1, :], tbt_ref[1:2, :]
        tw, th = tbt_ref[2:3, :], tbt_ref[3:4, :]

        l1 = (jnp.abs(pcx - tcx) + jnp.abs(pcy - tcy)
              + jnp.abs(pw - tw) + jnp.abs(ph - th))

        ax1, ax2 = pcx - 0.5 * pw, pcx + 0.5 * pw
        ay1, ay2 = pcy - 0.5 * ph, pcy + 0.5 * ph
        bx1, bx2 = tcx - 0.5 * tw, tcx + 0.5 * tw
        by1, by2 = tcy - 0.5 * th, tcy + 0.5 * th
        area_a = (ax2 - ax1) * (ay2 - ay1)
        area_b = (bx2 - bx1) * (by2 - by1)
        iw = jnp.maximum(jnp.minimum(ax2, bx2) - jnp.maximum(ax1, bx1), 0.0)
        ih = jnp.maximum(jnp.minimum(ay2, by2) - jnp.maximum(ay1, by1), 0.0)
        inter = iw * ih
        union = area_a + area_b - inter
        iou = inter / union
        ew = jnp.maximum(bx2, ax2) - jnp.minimum(bx1, ax1)
        eh = jnp.maximum(by2, ay2) - jnp.minimum(by1, ay1)
        area_e = ew * eh
        giou = iou - (area_e - union) / area_e

        out_ref[...] = (_COST_MASK * cost_mask + _COST_DICE * cost_dice
                        + _COST_BOX * l1 - _COST_GIOU * giou)


def kernel(pred_masks, tgt_masks, pred_boxes, tgt_boxes, point_coords):
    B, Q, H, W = pred_masks.shape
    T = tgt_masks.shape[1]
    P = point_coords.shape[1]
    hw = H * W
    wp = W + 2
    hwp = (H + 2) * wp
    f32 = jnp.float32

    zq = jnp.zeros((B, _QP - Q, H, W), f32)
    zt = jnp.zeros((B, _QP - T, H, W), f32)
    cat4 = jnp.concatenate([pred_masks, zq, tgt_masks, zt], axis=1)
    cat5 = cat4.reshape(B, _NPB, _NB, H, W)

    n_rk = H // _RH
    src = pl.pallas_call(
        _pack_kernel,
        grid=(B * _NPB, n_rk),
        in_specs=[pl.BlockSpec((None, None, _NB, _RH, W),
                               lambda g, k: (g // _NPB, g % _NPB, 0, k, 0))],
        out_specs=pl.BlockSpec((None, _RH * W, _NB), lambda g, k: (g, k, 0)),
        out_shape=jax.ShapeDtypeStruct((B * _NPB, hw, _NB), f32),
        compiler_params=pltpu.CompilerParams(
            dimension_semantics=("parallel", "arbitrary"),
            vmem_limit_bytes=56 * 1024 * 1024,
        ),
    )(cat5)
    src4 = src.reshape(B * _NPB, hw, 1, _NB)

    px = point_coords[..., 0] * W - 0.5
    py = point_coords[..., 1] * H - 0.5
    x0 = jnp.floor(px)
    y0 = jnp.floor(py)
    wxa = (px - x0).reshape(B, 1, P)
    wya = (py - y0).reshape(B, 1, P)
    ix = x0.astype(jnp.int32) + 1
    iy = y0.astype(jnp.int32) + 1
    idx = (iy * wp + ix).reshape(B, 1, P)

    sampled = pl.pallas_call(
        lambda *refs: _sample_kernel(*refs, h=H, w=W),
        grid=(B * _NPB,),
        in_specs=[
            pl.BlockSpec((None, 1, P), lambda g: (g // _NPB, 0, 0), memory_space=pltpu.SMEM),
            pl.BlockSpec((None, 1, P), lambda g: (g // _NPB, 0, 0), memory_space=pltpu.SMEM),
            pl.BlockSpec((None, 1, P), lambda g: (g // _NPB, 0, 0), memory_space=pltpu.SMEM),
            pl.BlockSpec(memory_space=pl.ANY),
        ],
        out_specs=pl.BlockSpec((None, P, 1, _NB), lambda g: (g // _NPB, 0, 0, g % _NPB)),
        out_shape=jax.ShapeDtypeStruct((B, P, 1, _N2), f32),
        scratch_shapes=[
            pltpu.VMEM((hwp, 1, _NB), f32),
            pltpu.SemaphoreType.DMA,
        ],
        compiler_params=pltpu.CompilerParams(
            dimension_semantics=("parallel",),
            vmem_limit_bytes=56 * 1024 * 1024,
        ),
    )(idx, wxa, wya, src4)

    samp = sampled.reshape(B, P, _N2)

    pb = jnp.zeros((B, _QP, 4), f32).at[:, :Q, :].set(pred_boxes)
    tbt = jnp.zeros((B, 4, _QP), f32).at[:, :, :T].set(tgt_boxes.transpose(0, 2, 1))

    pc = P // _KC
    out = pl.pallas_call(
        _cost_kernel,
        grid=(B, _KC),
        in_specs=[
            pl.BlockSpec((None, pc, _N2), lambda b, k: (b, k, 0)),
            pl.BlockSpec((None, _QP, 4), lambda b, k: (b, 0, 0)),
            pl.BlockSpec((None, 4, _QP), lambda b, k: (b, 0, 0)),
        ],
        out_specs=pl.BlockSpec((None, _QP, _QP), lambda b, k: (b, 0, 0)),
        out_shape=jax.ShapeDtypeStruct((B, _QP, _QP), f32),
        scratch_shapes=[
            pltpu.VMEM((_QP, _QP), f32),
            pltpu.VMEM((_QP, _QP + _NB), f32),
            pltpu.VMEM((_QP, _NB), f32),
            pltpu.VMEM((_NB, _QP), f32),
        ],
        compiler_params=pltpu.CompilerParams(
            dimension_semantics=("parallel", "arbitrary"),
            vmem_limit_bytes=56 * 1024 * 1024,
        ),
    )(samp, pb, tbt)

    return out[:, :Q, :T]


# speedup vs baseline: 3.4247x; 1.0074x over previous
import jax
import jax.numpy as jnp
from jax.experimental import pallas as pl
from jax.experimental.pallas import tpu as pltpu

_COST_MASK = 5.0
_COST_DICE = 5.0
_COST_BOX = 5.0
_COST_GIOU = 2.0

_NB = 128
_NPB = 6
_N2 = _NB * _NPB
_QP = 384
_U = 8
_KC = 8
_RH = 32


def _pack_kernel(x_ref, o_ref):
    w = x_ref.shape[2]
    ident = jnp.eye(128, dtype=jnp.float32)
    dn = (((0,), (0,)), ((), ()))
    for r in range(_RH):
        piece = x_ref[:, r, :]
        o_ref[r * w:(r + 1) * w, :] = jax.lax.dot_general(
            piece, ident, dn, preferred_element_type=jnp.float32)


def _sample_kernel(idx_ref, wx_ref, wy_ref, src_hbm, out_ref, buf, sem, *, h, w):
    g = pl.program_id(0)
    wp = w + 2
    zrow = jnp.zeros((_NB,), jnp.float32)
    buf[pl.ds(0, wp), 0, :] = jnp.zeros((wp, _NB), jnp.float32)
    buf[pl.ds((h + 1) * wp, wp), 0, :] = jnp.zeros((wp, _NB), jnp.float32)

    def zbody(y, _):
        buf[y * wp, 0, :] = zrow
        buf[y * wp + wp - 1, 0, :] = zrow
        return ()
    jax.lax.fori_loop(1, h + 1, zbody, ())

    for y in range(h):
        pltpu.make_async_copy(src_hbm.at[g, pl.ds(y * w, w)],
                              buf.at[pl.ds((y + 1) * wp + 1, w)], sem).start()
    for y in range(h):
        pltpu.make_async_copy(src_hbm.at[g, pl.ds(y * w, w)],
                              buf.at[pl.ds((y + 1) * wp + 1, w)], sem).wait()

    npts = out_ref.shape[0]

    def body(c, _):
        base = c * _U
        rows = []
        for u in range(_U):
            p = base + u
            i00 = idx_ref[0, p]
            wxs = wx_ref[0, p]
            wys = wy_ref[0, p]
            v00 = buf[i00, 0]
            v01 = buf[i00 + 1, 0]
            v10 = buf[i00 + wp, 0]
            v11 = buf[i00 + wp + 1, 0]
            r0 = v00 + wxs * (v01 - v00)
            r1 = v10 + wxs * (v11 - v10)
            rows.append(r0 + wys * (r1 - r0))
        out_ref[pl.ds(base, _U), :] = jnp.stack(rows, axis=0)
        return ()

    jax.lax.fori_loop(0, npts // _U, body, ())


def _cost_kernel(samp_ref, pb_ref, tbt_ref, out_ref, a_acc, sd_acc, neg_acc, tms_acc):
    k = pl.program_id(1)
    pc = samp_ref.shape[0]

    @pl.when(k == 0)
    def _():
        a_acc[...] = jnp.zeros_like(a_acc)
        sd_acc[...] = jnp.zeros_like(sd_acc)
        neg_acc[...] = jnp.zeros_like(neg_acc)
        tms_acc[...] = jnp.zeros_like(tms_acc)

    om = samp_ref[:, 0:_QP]
    tm = samp_ref[:, _QP:2 * _QP]
    e = jnp.exp(-jnp.abs(om))
    s_abs = 1.0 / (1.0 + e)
    s = jnp.where(om >= 0.0, s_abs, 1.0 - s_abs)
    neg = jnp.maximum(om, 0.0) + jnp.log(1.0 + e)

    ones = jnp.ones((pc, _NB), jnp.float32)
    rhs = jnp.concatenate([tm, ones], axis=1)
    dn = (((0,), (0,)), ((), ()))
    a_acc[...] += jax.lax.dot_general(om, tm, dn, preferred_element_type=jnp.float32)
    sd_acc[...] += jax.lax.dot_general(s, rhs, dn, preferred_element_type=jnp.float32)
    neg_acc[...] += jax.lax.dot_general(neg, ones, dn, preferred_element_type=jnp.float32)
    tms_acc[...] += jax.lax.dot_general(ones, tm, dn, preferred_element_type=jnp.float32)

    @pl.when(k == _KC - 1)
    def _():
        p_total = float(pc * _KC)
        a = a_acc[...]
        sd = sd_acc[:, 0:_QP]
        ssum = pltpu.repeat(sd_acc[:, _QP:_QP + _NB], _QP // _NB, axis=1)
        negs = pltpu.repeat(neg_acc[...], _QP // _NB, axis=1)
        tms = pltpu.repeat(tms_acc[...], _QP // _NB, axis=0)

        cost_mask = (negs - a) * (1.0 / p_total)
        cost_dice = 1.0 - (2.0 * sd + 1.0) / (ssum + tms + 1.0)

        pcx, pcy = pb_ref[:, 0:1], pb_ref[:, 1:2]
        pw, ph = pb_ref[:, 2:3], pb_ref[:, 3:4]
        tcx, tcy = tbt_ref[0:1, :], tbt_ref[1:2, :]
        tw, th = tbt_ref[2:3, :], tbt_ref[3:4, :]

        l1 = (jnp.abs(pcx - tcx) + jnp.abs(pcy - tcy)
              + jnp.abs(pw - tw) + jnp.abs(ph - th))

        ax1, ax2 = pcx - 0.5 * pw, pcx + 0.5 * pw
        ay1, ay2 = pcy - 0.5 * ph, pcy + 0.5 * ph
        bx1, bx2 = tcx - 0.5 * tw, tcx + 0.5 * tw
        by1, by2 = tcy - 0.5 * th, tcy + 0.5 * th
        area_a = (ax2 - ax1) * (ay2 - ay1)
        area_b = (bx2 - bx1) * (by2 - by1)
        iw = jnp.maximum(jnp.minimum(ax2, bx2) - jnp.maximum(ax1, bx1), 0.0)
        ih = jnp.maximum(jnp.minimum(ay2, by2) - jnp.maximum(ay1, by1), 0.0)
        inter = iw * ih
        union = area_a + area_b - inter
        iou = inter / union
        ew = jnp.maximum(bx2, ax2) - jnp.minimum(bx1, ax1)
        eh = jnp.maximum(by2, ay2) - jnp.minimum(by1, ay1)
        area_e = ew * eh
        giou = iou - (area_e - union) / area_e

        out_ref[...] = (_COST_MASK * cost_mask + _COST_DICE * cost_dice
                        + _COST_BOX * l1 - _COST_GIOU * giou)


def kernel(pred_masks, tgt_masks, pred_boxes, tgt_boxes, point_coords):
    B, Q, H, W = pred_masks.shape
    T = tgt_masks.shape[1]
    P = point_coords.shape[1]
    hw = H * W
    wp = W + 2
    hwp = (H + 2) * wp
    f32 = jnp.float32

    zq = jnp.zeros((B, _QP - Q, H, W), f32)
    zt = jnp.zeros((B, _QP - T, H, W), f32)
    cat4 = jnp.concatenate([pred_masks, zq, tgt_masks, zt], axis=1)
    cat5 = cat4.reshape(B, _NPB, _NB, H, W)

    n_rk = H // _RH
    src = pl.pallas_call(
        _pack_kernel,
        grid=(B * _NPB, n_rk),
        in_specs=[pl.BlockSpec((None, None, _NB, _RH, W),
                               lambda g, k: (g // _NPB, g % _NPB, 0, k, 0))],
        out_specs=pl.BlockSpec((None, _RH * W, _NB), lambda g, k: (g, k, 0)),
        out_shape=jax.ShapeDtypeStruct((B * _NPB, hw, _NB), f32),
        compiler_params=pltpu.CompilerParams(
            dimension_semantics=("parallel", "arbitrary"),
            vmem_limit_bytes=56 * 1024 * 1024,
        ),
    )(cat5)
    src4 = src.reshape(B * _NPB, hw, 1, _NB)

    px = point_coords[..., 0] * W - 0.5
    py = point_coords[..., 1] * H - 0.5
    x0 = jnp.floor(px)
    y0 = jnp.floor(py)
    wxa = (px - x0).reshape(B, 1, P)
    wya = (py - y0).reshape(B, 1, P)
    ix = x0.astype(jnp.int32) + 1
    iy = y0.astype(jnp.int32) + 1
    idx = (iy * wp + ix).reshape(B, 1, P)

    sampled = pl.pallas_call(
        lambda *refs: _sample_kernel(*refs, h=H, w=W),
        grid=(B * _NPB,),
        in_specs=[
            pl.BlockSpec((None, 1, P), lambda g: (g // _NPB, 0, 0), memory_space=pltpu.SMEM),
            pl.BlockSpec((None, 1, P), lambda g: (g // _NPB, 0, 0), memory_space=pltpu.SMEM),
            pl.BlockSpec((None, 1, P), lambda g: (g // _NPB, 0, 0), memory_space=pltpu.SMEM),
            pl.BlockSpec(memory_space=pl.ANY),
        ],
        out_specs=pl.BlockSpec((None, P, _NB), lambda g: (g // _NPB, 0, g % _NPB)),
        out_shape=jax.ShapeDtypeStruct((B, P, _N2), f32),
        scratch_shapes=[
            pltpu.VMEM((hwp, 1, _NB), f32),
            pltpu.SemaphoreType.DMA,
        ],
        compiler_params=pltpu.CompilerParams(
            dimension_semantics=("parallel",),
            vmem_limit_bytes=56 * 1024 * 1024,
        ),
    )(idx, wxa, wya, src4)

    samp = sampled

    pb = jnp.zeros((B, _QP, 4), f32).at[:, :Q, :].set(pred_boxes)
    tbt = jnp.zeros((B, 4, _QP), f32).at[:, :, :T].set(tgt_boxes.transpose(0, 2, 1))

    pc = P // _KC
    out = pl.pallas_call(
        _cost_kernel,
        grid=(B, _KC),
        in_specs=[
            pl.BlockSpec((None, pc, _N2), lambda b, k: (b, k, 0)),
            pl.BlockSpec((None, _QP, 4), lambda b, k: (b, 0, 0)),
            pl.BlockSpec((None, 4, _QP), lambda b, k: (b, 0, 0)),
        ],
        out_specs=pl.BlockSpec((None, _QP, _QP), lambda b, k: (b, 0, 0)),
        out_shape=jax.ShapeDtypeStruct((B, _QP, _QP), f32),
        scratch_shapes=[
            pltpu.VMEM((_QP, _QP), f32),
            pltpu.VMEM((_QP, _QP + _NB), f32),
            pltpu.VMEM((_QP, _NB), f32),
            pltpu.VMEM((_NB, _QP), f32),
        ],
        compiler_params=pltpu.CompilerParams(
            dimension_semantics=("parallel", "arbitrary"),
            vmem_limit_bytes=56 * 1024 * 1024,
        ),
    )(samp, pb, tbt)

    return out[:, :Q, :T]


# speedup vs baseline: 3.8454x; 1.1228x over previous
import jax
import jax.numpy as jnp
from jax.experimental import pallas as pl
from jax.experimental.pallas import tpu as pltpu

_COST_MASK = 5.0
_COST_DICE = 5.0
_COST_BOX = 5.0
_COST_GIOU = 2.0

_NB = 128
_NPB = 6
_N2 = _NB * _NPB
_QP = 384
_U = 16
_KC = 8
_RH = 64


def _pack_kernel(x_ref, o_ref):
    w = x_ref.shape[2]
    ident = jnp.eye(128, dtype=jnp.float32)
    dn = (((0,), (0,)), ((), ()))
    for r in range(_RH):
        piece = x_ref[:, r, :]
        o_ref[r * w:(r + 1) * w, :] = jax.lax.dot_general(
            piece, ident, dn, preferred_element_type=jnp.float32)


def _sample_kernel(idx_ref, wx_ref, wy_ref, src_hbm, out_ref, buf, sem, *, h, w):
    g = pl.program_id(0)
    wp = w + 2
    for y in range(h):
        pltpu.make_async_copy(src_hbm.at[g, pl.ds(y * w, w)],
                              buf.at[pl.ds((y + 1) * wp + 1, w)], sem).start()

    zrow = jnp.zeros((_NB,), jnp.float32)
    buf[pl.ds(0, wp), 0, :] = jnp.zeros((wp, _NB), jnp.float32)
    buf[pl.ds((h + 1) * wp, wp), 0, :] = jnp.zeros((wp, _NB), jnp.float32)

    def zbody(y, _):
        buf[y * wp, 0, :] = zrow
        buf[y * wp + wp - 1, 0, :] = zrow
        return ()
    jax.lax.fori_loop(1, h + 1, zbody, ())

    for y in range(h):
        pltpu.make_async_copy(src_hbm.at[g, pl.ds(y * w, w)],
                              buf.at[pl.ds((y + 1) * wp + 1, w)], sem).wait()

    npts = out_ref.shape[0]

    def body(c, _):
        base = c * _U
        rows = []
        for u in range(_U):
            p = base + u
            i00 = idx_ref[0, p]
            wxs = wx_ref[0, p]
            wys = wy_ref[0, p]
            v00 = buf[i00, 0]
            v01 = buf[i00 + 1, 0]
            v10 = buf[i00 + wp, 0]
            v11 = buf[i00 + wp + 1, 0]
            r0 = v00 + wxs * (v01 - v00)
            r1 = v10 + wxs * (v11 - v10)
            rows.append(r0 + wys * (r1 - r0))
        out_ref[pl.ds(base, _U), :] = jnp.stack(rows, axis=0)
        return ()

    jax.lax.fori_loop(0, npts // _U, body, ())


def _cost_kernel(samp_ref, pb_ref, tbt_ref, out_ref, a_acc, sd_acc, neg_acc, tms_acc):
    k = pl.program_id(1)
    pc = samp_ref.shape[0]

    @pl.when(k == 0)
    def _():
        a_acc[...] = jnp.zeros_like(a_acc)
        sd_acc[...] = jnp.zeros_like(sd_acc)
        neg_acc[...] = jnp.zeros_like(neg_acc)
        tms_acc[...] = jnp.zeros_like(tms_acc)

    om = samp_ref[:, 0:_QP]
    tm = samp_ref[:, _QP:2 * _QP]
    e = jnp.exp(-jnp.abs(om))
    s_abs = 1.0 / (1.0 + e)
    s = jnp.where(om >= 0.0, s_abs, 1.0 - s_abs)
    neg = jnp.maximum(om, 0.0) + jnp.log(1.0 + e)

    ones = jnp.ones((pc, _NB), jnp.float32)
    rhs = jnp.concatenate([tm, ones], axis=1)
    dn = (((0,), (0,)), ((), ()))
    a_acc[...] += jax.lax.dot_general(om, tm, dn, preferred_element_type=jnp.float32)
    sd_acc[...] += jax.lax.dot_general(s, rhs, dn, preferred_element_type=jnp.float32)
    neg_acc[...] += jax.lax.dot_general(neg, ones, dn, preferred_element_type=jnp.float32)
    tms_acc[...] += jax.lax.dot_general(ones, tm, dn, preferred_element_type=jnp.float32)

    @pl.when(k == _KC - 1)
    def _():
        p_total = float(pc * _KC)
        a = a_acc[...]
        sd = sd_acc[:, 0:_QP]
        ssum = pltpu.repeat(sd_acc[:, _QP:_QP + _NB], _QP // _NB, axis=1)
        negs = pltpu.repeat(neg_acc[...], _QP // _NB, axis=1)
        tms = pltpu.repeat(tms_acc[...], _QP // _NB, axis=0)

        cost_mask = (negs - a) * (1.0 / p_total)
        cost_dice = 1.0 - (2.0 * sd + 1.0) / (ssum + tms + 1.0)

        pcx, pcy = pb_ref[:, 0:1], pb_ref[:, 1:2]
        pw, ph = pb_ref[:, 2:3], pb_ref[:, 3:4]
        tcx, tcy = tbt_ref[0:1, :], tbt_ref[1:2, :]
        tw, th = tbt_ref[2:3, :], tbt_ref[3:4, :]

        l1 = (jnp.abs(pcx - tcx) + jnp.abs(pcy - tcy)
              + jnp.abs(pw - tw) + jnp.abs(ph - th))

        ax1, ax2 = pcx - 0.5 * pw, pcx + 0.5 * pw
        ay1, ay2 = pcy - 0.5 * ph, pcy + 0.5 * ph
        bx1, bx2 = tcx - 0.5 * tw, tcx + 0.5 * tw
        by1, by2 = tcy - 0.5 * th, tcy + 0.5 * th
        area_a = (ax2 - ax1) * (ay2 - ay1)
        area_b = (bx2 - bx1) * (by2 - by1)
        iw = jnp.maximum(jnp.minimum(ax2, bx2) - jnp.maximum(ax1, bx1), 0.0)
        ih = jnp.maximum(jnp.minimum(ay2, by2) - jnp.maximum(ay1, by1), 0.0)
        inter = iw * ih
        union = area_a + area_b - inter
        iou = inter / union
        ew = jnp.maximum(bx2, ax2) - jnp.minimum(bx1, ax1)
        eh = jnp.maximum(by2, ay2) - jnp.minimum(by1, ay1)
        area_e = ew * eh
        giou = iou - (area_e - union) / area_e

        out_ref[...] = (_COST_MASK * cost_mask + _COST_DICE * cost_dice
                        + _COST_BOX * l1 - _COST_GIOU * giou)


def kernel(pred_masks, tgt_masks, pred_boxes, tgt_boxes, point_coords):
    B, Q, H, W = pred_masks.shape
    T = tgt_masks.shape[1]
    P = point_coords.shape[1]
    hw = H * W
    wp = W + 2
    hwp = (H + 2) * wp
    f32 = jnp.float32

    zq = jnp.zeros((B, _QP - Q, H, W), f32)
    zt = jnp.zeros((B, _QP - T, H, W), f32)
    cat4 = jnp.concatenate([pred_masks, zq, tgt_masks, zt], axis=1)
    cat5 = cat4.reshape(B, _NPB, _NB, H, W)

    n_rk = H // _RH
    src = pl.pallas_call(
        _pack_kernel,
        grid=(B * _NPB, n_rk),
        in_specs=[pl.BlockSpec((None, None, _NB, _RH, W),
                               lambda g, k: (g // _NPB, g % _NPB, 0, k, 0))],
        out_specs=pl.BlockSpec((None, _RH * W, _NB), lambda g, k: (g, k, 0)),
        out_shape=jax.ShapeDtypeStruct((B * _NPB, hw, _NB), f32),
        compiler_params=pltpu.CompilerParams(
            dimension_semantics=("parallel", "arbitrary"),
            vmem_limit_bytes=56 * 1024 * 1024,
        ),
    )(cat5)
    src4 = src.reshape(B * _NPB, hw, 1, _NB)

    px = point_coords[..., 0] * W - 0.5
    py = point_coords[..., 1] * H - 0.5
    x0 = jnp.floor(px)
    y0 = jnp.floor(py)
    wxa = (px - x0).reshape(B, 1, P)
    wya = (py - y0).reshape(B, 1, P)
    ix = x0.astype(jnp.int32) + 1
    iy = y0.astype(jnp.int32) + 1
    idx = (iy * wp + ix).reshape(B, 1, P)

    sampled = pl.pallas_call(
        lambda *refs: _sample_kernel(*refs, h=H, w=W),
        grid=(B * _NPB,),
        in_specs=[
            pl.BlockSpec((None, 1, P), lambda g: (g // _NPB, 0, 0), memory_space=pltpu.SMEM),
            pl.BlockSpec((None, 1, P), lambda g: (g // _NPB, 0, 0), memory_space=pltpu.SMEM),
            pl.BlockSpec((None, 1, P), lambda g: (g // _NPB, 0, 0), memory_space=pltpu.SMEM),
            pl.BlockSpec(memory_space=pl.ANY),
        ],
        out_specs=pl.BlockSpec((None, P, _NB), lambda g: (g // _NPB, 0, g % _NPB)),
        out_shape=jax.ShapeDtypeStruct((B, P, _N2), f32),
        scratch_shapes=[
            pltpu.VMEM((hwp, 1, _NB), f32),
            pltpu.SemaphoreType.DMA,
        ],
        compiler_params=pltpu.CompilerParams(
            dimension_semantics=("parallel",),
            vmem_limit_bytes=56 * 1024 * 1024,
        ),
    )(idx, wxa, wya, src4)

    pb = jnp.zeros((B, _QP, 4), f32).at[:, :Q, :].set(pred_boxes)
    tbt = jnp.zeros((B, 4, _QP), f32).at[:, :, :T].set(tgt_boxes.transpose(0, 2, 1))

    pc = P // _KC
    out = pl.pallas_call(
        _cost_kernel,
        grid=(B, _KC),
        in_specs=[
            pl.BlockSpec((None, pc, _N2), lambda b, k: (b, k, 0)),
            pl.BlockSpec((None, _QP, 4), lambda b, k: (b, 0, 0)),
            pl.BlockSpec((None, 4, _QP), lambda b, k: (b, 0, 0)),
        ],
        out_specs=pl.BlockSpec((None, _QP, _QP), lambda b, k: (b, 0, 0)),
        out_shape=jax.ShapeDtypeStruct((B, _QP, _QP), f32),
        scratch_shapes=[
            pltpu.VMEM((_QP, _QP), f32),
            pltpu.VMEM((_QP, _QP + _NB), f32),
            pltpu.VMEM((_QP, _NB), f32),
            pltpu.VMEM((_NB, _QP), f32),
        ],
        compiler_params=pltpu.CompilerParams(
            dimension_semantics=("parallel", "arbitrary"),
            vmem_limit_bytes=56 * 1024 * 1024,
        ),
    )(sampled, pb, tbt)

    return out[:, :Q, :T]
